# baseline (speedup 1.0000x reference)
import functools

import jax
import jax.numpy as jnp
from jax import lax
from jax.experimental import pallas as pl
from jax.experimental.pallas import tpu as pltpu
from jax.experimental.pallas import tpu_sc as plsc

_L = 16


@functools.cache
def _make_fused(batch, vocab, dim, hidden):
    info = plsc.get_sparse_core_info()
    nc, ns = info.num_cores, info.num_subcores
    nw = nc * ns
    b_per_w = batch // nw
    n_chunks = b_per_w // _L
    assert batch == nw * n_chunks * _L and dim == _L

    mesh = plsc.VectorSubcoreMesh(core_axis_name="c", subcore_axis_name="s")

    @functools.partial(
        pl.kernel,
        mesh=mesh,
        compiler_params=pltpu.CompilerParams(
            use_tc_tiling_on_sc=False, needs_layout_passes=False
        ),
        out_type=jax.ShapeDtypeStruct((batch * dim,), jnp.float32),
        scratch_types=[
            pltpu.VMEM((vocab * dim,), jnp.float32),
            pltpu.VMEM((hidden * dim,), jnp.float32),
            pltpu.VMEM((hidden,), jnp.float32),
            pltpu.VMEM((dim * hidden,), jnp.float32),
            pltpu.VMEM((dim,), jnp.float32),
            pltpu.VMEM((_L * dim,), jnp.float32),
            pltpu.VMEM((b_per_w,), jnp.int32),
            pltpu.VMEM((b_per_w * dim,), jnp.float32),
            pltpu.SemaphoreType.DMA,
        ],
    )
    def fused(tab_hbm, w1_hbm, b1_hbm, w2_hbm, b2_hbm, idx_hbm, out_hbm,
              tab_v, w1_v, b1_v, w2_v, b2_v, tout_v, idx_v, out_v, sem):
        wid = lax.axis_index("s") * nc + lax.axis_index("c")
        base = wid * b_per_w

        copies = [
            pltpu.async_copy(tab_hbm, tab_v, sem),
            pltpu.async_copy(w1_hbm, w1_v, sem),
            pltpu.async_copy(b1_hbm, b1_v, sem),
            pltpu.async_copy(w2_hbm, w2_v, sem),
            pltpu.async_copy(b2_hbm, b2_v, sem),
            pltpu.async_copy(idx_hbm.at[pl.ds(base, b_per_w)], idx_v, sem),
        ]
        for cp in copies:
            cp.wait()

        iota = lax.iota(jnp.int32, _L)
        vbase = jnp.minimum(iota, vocab - 1) * dim

        t = [plsc.load_gather(tab_v, [vbase + d]) for d in range(dim)]

        b2row = b2_v[...]
        acc = [jnp.full((_L,), b2row[i]) for i in range(dim)]
        b1rows = [b1_v[pl.ds(half * _L, _L)] for half in range(hidden // _L)]
        for j in range(hidden):
            h = jnp.full((_L,), b1rows[j // _L][j % _L])
            w1row = w1_v[pl.ds(j * dim, dim)]
            for d in range(dim):
                h = h + t[d] * w1row[d]
            h = jnp.maximum(h, 0.0)
            for i in range(dim):
                w2row = w2_v[pl.ds(i * hidden + (j // _L) * _L, _L)]
                acc[i] = acc[i] + h * w2row[j % _L]
        tbase = iota * dim
        for i in range(dim):
            e = jnp.exp(acc[i] * 2.0)
            plsc.store_scatter(tout_v, [tbase + i], 1.0 - 2.0 / (e + 1.0))

        def chunk(c, carry):
            iv16 = idx_v[pl.ds(c * _L, _L)] * dim
            obase = (c * _L + iota) * dim
            for d in range(dim):
                vals = plsc.load_gather(tout_v, [iv16 + d])
                plsc.store_scatter(out_v, [obase + d], vals)
            return carry

        lax.fori_loop(0, n_chunks, chunk, 0)

        pltpu.sync_copy(out_v, out_hbm.at[pl.ds(base * dim, b_per_w * dim)])

    return fused


def kernel(device_type_id, table, W1, b1, W2, b2):
    vocab, dim = table.shape
    hidden = W1.shape[0]
    batch = device_type_id.shape[0]
    out_flat = _make_fused(batch, vocab, dim, hidden)(
        table.reshape(-1),
        W1.reshape(-1),
        b1,
        W2.reshape(-1),
        b2,
        device_type_id.astype(jnp.int32),
    )
    return out_flat.reshape(batch, dim)
